# baseline (speedup 1.0000x reference)
import functools

import jax
import jax.numpy as jnp
from jax import lax
from jax.experimental import pallas as pl
from jax.experimental.pallas import tpu as pltpu
from jax.experimental.pallas import tpu_sc as plsc

NUM_GRAPHS = 256

try:
    _info = plsc.get_sparse_core_info()
    _NC, _NS, _L = _info.num_cores, _info.num_subcores, _info.num_lanes
except Exception:
    _NC, _NS, _L = 2, 16, 16
_NW = _NC * _NS

_SC_PARAMS = pltpu.CompilerParams(needs_layout_passes=False)


def _make_edge_kernel(N, HN, E, CH):
    NCHT = E // CH
    NPS = NCHT // _NS
    NV = CH // _L
    mesh = plsc.VectorSubcoreMesh(core_axis_name="c", subcore_axis_name="s")

    @functools.partial(
        pl.kernel, mesh=mesh,
        out_type=jax.ShapeDtypeStruct((_NW, HN), jnp.float32),
        scratch_types=[
            pltpu.VMEM((N // 2,), jnp.int32),
            pltpu.VMEM((HN,), jnp.float32),
            pltpu.VMEM((2, CH), jnp.int32),
            pltpu.VMEM((2, CH), jnp.int32),
            pltpu.SemaphoreType.DMA,
            pltpu.SemaphoreType.DMA,
            pltpu.SemaphoreType.DMA,
        ],
        compiler_params=_SC_PARAMS,
    )
    def edge_k(xb_hbm, edge_hbm, part_hbm, xv, acc, e0, e1,
               sem_x, sem_e0, sem_e1):
        c = lax.axis_index("c")
        s = lax.axis_index("s")
        base_node = c * HN

        def col(j):
            return (s + j * _NS) * CH

        pltpu.async_copy(xb_hbm, xv, sem_x)
        pltpu.async_copy(edge_hbm.at[:, pl.ds(col(0), CH)], e0, sem_e0)

        def zero(i, z):
            for k in range(8):
                acc[pl.ds(i * 8 * _L + k * _L, _L)] = jnp.zeros(
                    (_L,), jnp.float32)
            return z

        lax.fori_loop(0, HN // (8 * _L), zero, 0)
        pltpu.make_async_copy(xb_hbm, xv, sem_x).wait()

        def do_chunk(ebuf):
            def vec(i, z):
                src = ebuf[0, pl.ds(i * _L, _L)]
                dst = ebuf[1, pl.ds(i * _L, _L)]
                w = plsc.load_gather(xv, [lax.shift_right_logical(src, 1)])
                odd = (src & 1) == 1
                bits = jnp.where(odd, w & jnp.int32(-65536),
                                 lax.shift_left(w, 16))
                v = lax.bitcast_convert_type(bits, jnp.float32)
                local = dst - base_node
                valid = local.astype(jnp.uint32) < jnp.uint32(HN)
                plsc.addupdate_scatter(acc, [local], v, mask=valid)
                return z
            lax.fori_loop(0, NV, vec, 0, unroll=8)

        def pair(p, carry):
            j0 = 2 * p
            j1 = j0 + 1

            @pl.when(j1 < NPS)
            def _():
                pltpu.async_copy(edge_hbm.at[:, pl.ds(col(j1), CH)], e1,
                                 sem_e1)
            pltpu.make_async_copy(edge_hbm.at[:, pl.ds(col(j0), CH)], e0,
                                  sem_e0).wait()
            do_chunk(e0)

            @pl.when(j0 + 2 < NPS)
            def _():
                pltpu.async_copy(edge_hbm.at[:, pl.ds(col(j0 + 2), CH)], e0,
                                 sem_e0)

            @pl.when(j1 < NPS)
            def _():
                pltpu.make_async_copy(edge_hbm.at[:, pl.ds(col(j1), CH)], e1,
                                      sem_e1).wait()
                do_chunk(e1)
            return carry

        lax.fori_loop(0, (NPS + 1) // 2, pair, 0)
        pltpu.sync_copy(acc, part_hbm.at[c * _NS + s])

    return edge_k


def _tc_body(nblk, nhblk, B, part_ref, x_ref, batch_ref, wl_ref, bl_ref,
             wr_ref, wlin_ref, blin_ref, out_ref, sums, counts):
    i = pl.program_id(0)

    @pl.when(i == 0)
    def _init():
        sums[...] = jnp.zeros_like(sums)
        counts[...] = jnp.zeros_like(counts)

    part = part_ref[...]
    agg0 = jnp.sum(part[:_NS], axis=0)
    agg1 = jnp.sum(part[_NS:], axis=0)
    agg = jnp.where(i < nhblk, agg0, agg1)
    xb = x_ref[0, :]
    h = jnp.maximum(
        agg[:, None] * wl_ref[0, :][None, :]
        + xb[:, None] * wr_ref[0, :][None, :]
        + bl_ref[0, :][None, :], 0.0)
    bb = batch_ref[0, :]
    gid = lax.broadcasted_iota(jnp.int32, (NUM_GRAPHS, 1), 0)
    onehot = (bb[None, :] == gid).astype(jnp.bfloat16)
    sums[...] += jnp.dot(onehot, h.astype(jnp.bfloat16),
                         preferred_element_type=jnp.float32)
    cnt = jnp.sum(onehot.astype(jnp.float32), axis=1, keepdims=True)
    counts[...] += jnp.broadcast_to(cnt, counts.shape)

    @pl.when(i == nblk - 1)
    def _fin():
        pooled = sums[...] / jnp.maximum(counts[...], 1.0)
        out_ref[...] = (jnp.dot(pooled, wlin_ref[...],
                                preferred_element_type=jnp.float32)
                        + blin_ref[0, :][None, :])


def _make_tc(N_pad, HN, H, OUT, B):
    nblk = N_pad // B
    nhblk = HN // B
    return pl.pallas_call(
        functools.partial(_tc_body, nblk, nhblk, B),
        grid=(nblk,),
        in_specs=[
            pl.BlockSpec((_NW, B), lambda i: (0, i % nhblk)),
            pl.BlockSpec((1, B), lambda i: (0, i)),
            pl.BlockSpec((1, B), lambda i: (0, i)),
            pl.BlockSpec((1, H), lambda i: (0, 0)),
            pl.BlockSpec((1, H), lambda i: (0, 0)),
            pl.BlockSpec((1, H), lambda i: (0, 0)),
            pl.BlockSpec((H, OUT), lambda i: (0, 0)),
            pl.BlockSpec((1, OUT), lambda i: (0, 0)),
        ],
        out_specs=pl.BlockSpec((NUM_GRAPHS, OUT), lambda i: (0, 0)),
        out_shape=jax.ShapeDtypeStruct((NUM_GRAPHS, OUT), jnp.float32),
        scratch_shapes=[
            pltpu.VMEM((NUM_GRAPHS, H), jnp.float32),
            pltpu.VMEM((NUM_GRAPHS, H), jnp.float32),
        ],
    )


def kernel(x, edge_index, batch, W_l, b_l, W_r, W_lin, b_lin):
    N = x.shape[0]
    E = edge_index.shape[1]
    H = W_l.shape[0]
    OUT = W_lin.shape[0]
    CH = 3200
    B = 2048
    N_pad = 102400
    HN = N_pad // 2

    xb = lax.bitcast_convert_type(
        x.reshape(N // 2, 2).astype(jnp.bfloat16), jnp.int32)
    partials = _make_edge_kernel(N, HN, E, CH)(xb, edge_index)
    x_pad = jnp.pad(x.reshape(1, N), ((0, 0), (0, N_pad - N)))
    batch_pad = jnp.pad(batch.reshape(1, N), ((0, 0), (0, N_pad - N)),
                        constant_values=NUM_GRAPHS)
    out = _make_tc(N_pad, HN, H, OUT, B)(
        partials,
        x_pad,
        batch_pad,
        W_l.reshape(1, H),
        b_l.reshape(1, H),
        W_r.reshape(1, H),
        W_lin.T,
        b_lin.reshape(1, OUT),
    )
    return out


# speedup vs baseline: 1.9581x; 1.9581x over previous
import functools

import jax
import jax.numpy as jnp
from jax import lax
from jax.experimental import pallas as pl
from jax.experimental.pallas import tpu as pltpu
from jax.experimental.pallas import tpu_sc as plsc

NUM_GRAPHS = 256

try:
    _info = plsc.get_sparse_core_info()
    _NC, _NS, _L = _info.num_cores, _info.num_subcores, _info.num_lanes
except Exception:
    _NC, _NS, _L = 2, 16, 16
_NW = _NC * _NS

_SC_PARAMS = pltpu.CompilerParams(needs_layout_passes=False)


def _worker_id():
    return lax.axis_index("s") * _NC + lax.axis_index("c")


def _make_edge_kernel(N, N_pad, E, CH):
    NCHT = E // CH
    NV = CH // _L
    mesh = plsc.VectorSubcoreMesh(core_axis_name="c", subcore_axis_name="s")

    @functools.partial(
        pl.kernel, mesh=mesh,
        out_type=(
            jax.ShapeDtypeStruct((_NW, N_pad), jnp.float32),
            jax.ShapeDtypeStruct((E,), jnp.float32),
        ),
        scratch_types=[
            pltpu.VMEM((N_pad,), jnp.float32),
            pltpu.VMEM((2, CH), jnp.int32),
            pltpu.VMEM((2, CH), jnp.int32),
            pltpu.VMEM((CH,), jnp.float32),
            pltpu.VMEM((CH,), jnp.float32),
            pltpu.SemaphoreType.DMA,
            pltpu.SemaphoreType.DMA,
            pltpu.SemaphoreType.DMA,
            pltpu.SemaphoreType.DMA,
            pltpu.SemaphoreType.DMA,
        ],
        compiler_params=_SC_PARAMS,
    )
    def edge_k(x_hbm, edge_hbm, part_hbm, vals_hbm, big, e0, e1, v0, v1,
               sem_x, sem_e0, sem_e1, sem_v0, sem_v1):
        wid = _worker_id()
        cnt = (NCHT - wid + _NW - 1) // _NW

        def col(j):
            return (wid + j * _NW) * CH

        pltpu.async_copy(x_hbm, big.at[pl.ds(0, N)], sem_x)
        pltpu.async_copy(edge_hbm.at[:, pl.ds(col(0), CH)], e0, sem_e0)
        pltpu.make_async_copy(x_hbm, big.at[pl.ds(0, N)], sem_x).wait()

        def gather_chunk(ebuf, vbuf):
            @plsc.parallel_loop(0, NV, unroll=8)
            def _(i):
                idx = ebuf[0, pl.ds(i * _L, _L)]
                vbuf[pl.ds(i * _L, _L)] = plsc.load_gather(big, [idx])

        def g_pair(p, carry):
            j0 = 2 * p
            j1 = j0 + 1

            @pl.when(j1 < cnt)
            def _():
                pltpu.async_copy(edge_hbm.at[:, pl.ds(col(j1), CH)], e1,
                                 sem_e1)
            pltpu.make_async_copy(edge_hbm.at[:, pl.ds(col(j0), CH)], e0,
                                  sem_e0).wait()
            @pl.when(j0 >= 2)
            def _():
                pltpu.make_async_copy(v0, vals_hbm.at[pl.ds(col(j0), CH)],
                                      sem_v0).wait()
            gather_chunk(e0, v0)
            pltpu.async_copy(v0, vals_hbm.at[pl.ds(col(j0), CH)], sem_v0)

            @pl.when(j0 + 2 < cnt)
            def _():
                pltpu.async_copy(edge_hbm.at[:, pl.ds(col(j0 + 2), CH)], e0,
                                 sem_e0)

            @pl.when(j1 < cnt)
            def _():
                pltpu.make_async_copy(edge_hbm.at[:, pl.ds(col(j1), CH)], e1,
                                      sem_e1).wait()
                @pl.when(j1 >= 2)
                def _():
                    pltpu.make_async_copy(
                        v1, vals_hbm.at[pl.ds(col(j1), CH)], sem_v1).wait()
                gather_chunk(e1, v1)
                pltpu.async_copy(v1, vals_hbm.at[pl.ds(col(j1), CH)], sem_v1)
            return carry

        lax.fori_loop(0, (cnt + 1) // 2, g_pair, 0)
        pltpu.make_async_copy(v0, vals_hbm.at[pl.ds(0, CH)], sem_v0).wait()

        @pl.when(cnt >= 2)
        def _():
            pltpu.make_async_copy(v1, vals_hbm.at[pl.ds(0, CH)], sem_v1).wait()

        pltpu.async_copy(edge_hbm.at[:, pl.ds(col(0), CH)], e0, sem_e0)
        pltpu.async_copy(vals_hbm.at[pl.ds(col(0), CH)], v0, sem_v0)

        def zero(i, c):
            for k in range(8):
                big[pl.ds(i * 8 * _L + k * _L, _L)] = jnp.zeros(
                    (_L,), jnp.float32)
            return c

        lax.fori_loop(0, N_pad // (8 * _L), zero, 0)

        def scatter_chunk(ebuf, vbuf):
            def vec(i, c):
                idx = ebuf[1, pl.ds(i * _L, _L)]
                v = vbuf[pl.ds(i * _L, _L)]
                plsc.addupdate_scatter(big, [idx], v)
                return c
            lax.fori_loop(0, NV, vec, 0, unroll=8)

        def s_pair(p, carry):
            j0 = 2 * p
            j1 = j0 + 1

            @pl.when(j1 < cnt)
            def _():
                pltpu.async_copy(edge_hbm.at[:, pl.ds(col(j1), CH)], e1,
                                 sem_e1)
                pltpu.async_copy(vals_hbm.at[pl.ds(col(j1), CH)], v1, sem_v1)
            pltpu.make_async_copy(edge_hbm.at[:, pl.ds(col(j0), CH)], e0,
                                  sem_e0).wait()
            pltpu.make_async_copy(vals_hbm.at[pl.ds(col(j0), CH)], v0,
                                  sem_v0).wait()
            scatter_chunk(e0, v0)

            @pl.when(j0 + 2 < cnt)
            def _():
                pltpu.async_copy(edge_hbm.at[:, pl.ds(col(j0 + 2), CH)], e0,
                                 sem_e0)
                pltpu.async_copy(vals_hbm.at[pl.ds(col(j0 + 2), CH)], v0,
                                 sem_v0)

            @pl.when(j1 < cnt)
            def _():
                pltpu.make_async_copy(edge_hbm.at[:, pl.ds(col(j1), CH)], e1,
                                      sem_e1).wait()
                pltpu.make_async_copy(vals_hbm.at[pl.ds(col(j1), CH)], v1,
                                      sem_v1).wait()
                scatter_chunk(e1, v1)
            return carry

        lax.fori_loop(0, (cnt + 1) // 2, s_pair, 0)
        pltpu.sync_copy(big, part_hbm.at[wid])

    return edge_k


def _tc_body(nblk, N, B, part_ref, x_ref, batch_ref, wl_ref, bl_ref, wr_ref,
             wlin_ref, blin_ref, out_ref, sums, counts):
    i = pl.program_id(0)

    @pl.when(i == 0)
    def _init():
        sums[...] = jnp.zeros_like(sums)
        counts[...] = jnp.zeros_like(counts)

    pos = lax.broadcasted_iota(jnp.int32, (1, B), 1) + i * B
    valid = pos < N
    pos_c = lax.broadcasted_iota(jnp.int32, (B, 1), 0) + i * B
    valid_c = pos_c < N
    agg = jnp.sum(part_ref[...], axis=0)
    xb = x_ref[0, :]
    h = jnp.maximum(
        agg[:, None] * wl_ref[0, :][None, :]
        + xb[:, None] * wr_ref[0, :][None, :]
        + bl_ref[0, :][None, :], 0.0)
    h = jnp.where(valid_c, h, 0.0)
    bb = batch_ref[0, :]
    gid = lax.broadcasted_iota(jnp.int32, (NUM_GRAPHS, 1), 0)
    onehot = ((bb[None, :] == gid) & valid).astype(jnp.bfloat16)
    sums[...] += jnp.dot(onehot, h.astype(jnp.bfloat16),
                         preferred_element_type=jnp.float32)
    cnt = jnp.sum(onehot.astype(jnp.float32), axis=1, keepdims=True)
    counts[...] += jnp.broadcast_to(cnt, counts.shape)

    @pl.when(i == nblk - 1)
    def _fin():
        pooled = sums[...] / jnp.maximum(counts[...], 1.0)
        out_ref[...] = (jnp.dot(pooled, wlin_ref[...],
                                preferred_element_type=jnp.float32)
                        + blin_ref[0, :][None, :])


def _make_tc(N, N_pad, H, OUT, B):
    nblk = N_pad // B
    return pl.pallas_call(
        functools.partial(_tc_body, nblk, N, B),
        grid=(nblk,),
        in_specs=[
            pl.BlockSpec((_NW, B), lambda i: (0, i)),
            pl.BlockSpec((1, B), lambda i: (0, i)),
            pl.BlockSpec((1, B), lambda i: (0, i)),
            pl.BlockSpec((1, H), lambda i: (0, 0)),
            pl.BlockSpec((1, H), lambda i: (0, 0)),
            pl.BlockSpec((1, H), lambda i: (0, 0)),
            pl.BlockSpec((H, OUT), lambda i: (0, 0)),
            pl.BlockSpec((1, OUT), lambda i: (0, 0)),
        ],
        out_specs=pl.BlockSpec((NUM_GRAPHS, OUT), lambda i: (0, 0)),
        out_shape=jax.ShapeDtypeStruct((NUM_GRAPHS, OUT), jnp.float32),
        scratch_shapes=[
            pltpu.VMEM((NUM_GRAPHS, H), jnp.float32),
            pltpu.VMEM((NUM_GRAPHS, H), jnp.float32),
        ],
    )


def kernel(x, edge_index, batch, W_l, b_l, W_r, W_lin, b_lin):
    N = x.shape[0]
    E = edge_index.shape[1]
    H = W_l.shape[0]
    OUT = W_lin.shape[0]
    CH = 3200
    B = 4096
    N_pad = -(-N // B) * B

    xf = x.reshape(N)
    partials, _vals = _make_edge_kernel(N, N_pad, E, CH)(xf, edge_index)
    out = _make_tc(N, N_pad, H, OUT, B)(
        partials,
        x.reshape(1, N),
        batch.reshape(1, N),
        W_l.reshape(1, H),
        b_l.reshape(1, H),
        W_r.reshape(1, H),
        W_lin.T,
        b_lin.reshape(1, OUT),
    )
    return out


# speedup vs baseline: 2.0247x; 1.0340x over previous
import functools

import jax
import jax.numpy as jnp
from jax import lax
from jax.experimental import pallas as pl
from jax.experimental.pallas import tpu as pltpu
from jax.experimental.pallas import tpu_sc as plsc

NUM_GRAPHS = 256

try:
    _info = plsc.get_sparse_core_info()
    _NC, _NS, _L = _info.num_cores, _info.num_subcores, _info.num_lanes
except Exception:
    _NC, _NS, _L = 2, 16, 16
_NW = _NC * _NS

_SC_PARAMS = pltpu.CompilerParams(needs_layout_passes=False)


def _worker_id():
    return lax.axis_index("s") * _NC + lax.axis_index("c")


def _make_edge_kernel(N, N_pad, E, CH):
    NCHT = E // CH
    NV = CH // _L
    mesh = plsc.VectorSubcoreMesh(core_axis_name="c", subcore_axis_name="s")

    @functools.partial(
        pl.kernel, mesh=mesh,
        out_type=(
            jax.ShapeDtypeStruct((_NW, N_pad), jnp.float32),
            jax.ShapeDtypeStruct((E,), jnp.float32),
        ),
        scratch_types=[
            pltpu.VMEM((N_pad,), jnp.float32),
            pltpu.VMEM((2, CH), jnp.int32),
            pltpu.VMEM((2, CH), jnp.int32),
            pltpu.VMEM((CH,), jnp.float32),
            pltpu.VMEM((CH,), jnp.float32),
            pltpu.SemaphoreType.DMA,
            pltpu.SemaphoreType.DMA,
            pltpu.SemaphoreType.DMA,
            pltpu.SemaphoreType.DMA,
            pltpu.SemaphoreType.DMA,
        ],
        compiler_params=_SC_PARAMS,
    )
    def edge_k(x_hbm, edge_hbm, part_hbm, vals_hbm, big, e0, e1, v0, v1,
               sem_x, sem_e0, sem_e1, sem_v0, sem_v1):
        wid = _worker_id()
        cnt = (NCHT - wid + _NW - 1) // _NW

        def col(j):
            return (wid + j * _NW) * CH

        pltpu.async_copy(x_hbm, big.at[pl.ds(0, N)], sem_x)
        pltpu.async_copy(edge_hbm.at[:, pl.ds(col(0), CH)], e0, sem_e0)
        pltpu.make_async_copy(x_hbm, big.at[pl.ds(0, N)], sem_x).wait()

        def gather_chunk(ebuf, vbuf):
            @plsc.parallel_loop(0, NV, unroll=8)
            def _(i):
                idx = ebuf[0, pl.ds(i * _L, _L)]
                vbuf[pl.ds(i * _L, _L)] = plsc.load_gather(big, [idx])

        def g_pair(p, carry):
            j0 = 2 * p
            j1 = j0 + 1

            @pl.when(j1 < cnt)
            def _():
                pltpu.async_copy(edge_hbm.at[:, pl.ds(col(j1), CH)], e1,
                                 sem_e1)
            pltpu.make_async_copy(edge_hbm.at[:, pl.ds(col(j0), CH)], e0,
                                  sem_e0).wait()
            @pl.when(j0 >= 2)
            def _():
                pltpu.make_async_copy(v0, vals_hbm.at[pl.ds(col(j0), CH)],
                                      sem_v0).wait()
            gather_chunk(e0, v0)
            pltpu.async_copy(v0, vals_hbm.at[pl.ds(col(j0), CH)], sem_v0)

            @pl.when(j0 + 2 < cnt)
            def _():
                pltpu.async_copy(edge_hbm.at[:, pl.ds(col(j0 + 2), CH)], e0,
                                 sem_e0)

            @pl.when(j1 < cnt)
            def _():
                pltpu.make_async_copy(edge_hbm.at[:, pl.ds(col(j1), CH)], e1,
                                      sem_e1).wait()
                @pl.when(j1 >= 2)
                def _():
                    pltpu.make_async_copy(
                        v1, vals_hbm.at[pl.ds(col(j1), CH)], sem_v1).wait()
                gather_chunk(e1, v1)
                pltpu.async_copy(v1, vals_hbm.at[pl.ds(col(j1), CH)], sem_v1)
            return carry

        lax.fori_loop(0, (cnt + 1) // 2, g_pair, 0)
        pltpu.make_async_copy(v0, vals_hbm.at[pl.ds(0, CH)], sem_v0).wait()

        @pl.when(cnt >= 2)
        def _():
            pltpu.make_async_copy(v1, vals_hbm.at[pl.ds(0, CH)], sem_v1).wait()

        pltpu.async_copy(edge_hbm.at[:, pl.ds(col(0), CH)], e0, sem_e0)
        pltpu.async_copy(vals_hbm.at[pl.ds(col(0), CH)], v0, sem_v0)

        def zero(i, c):
            for k in range(8):
                big[pl.ds(i * 8 * _L + k * _L, _L)] = jnp.zeros(
                    (_L,), jnp.float32)
            return c

        lax.fori_loop(0, N_pad // (8 * _L), zero, 0)

        def scatter_chunk(ebuf, vbuf):
            def vec(i, c):
                idx = ebuf[1, pl.ds(i * _L, _L)]
                v = vbuf[pl.ds(i * _L, _L)]
                plsc.addupdate_scatter(big, [idx], v)
                return c
            lax.fori_loop(0, NV, vec, 0, unroll=8)

        def s_pair(p, carry):
            j0 = 2 * p
            j1 = j0 + 1

            @pl.when(j1 < cnt)
            def _():
                pltpu.async_copy(edge_hbm.at[:, pl.ds(col(j1), CH)], e1,
                                 sem_e1)
                pltpu.async_copy(vals_hbm.at[pl.ds(col(j1), CH)], v1, sem_v1)
            pltpu.make_async_copy(edge_hbm.at[:, pl.ds(col(j0), CH)], e0,
                                  sem_e0).wait()
            pltpu.make_async_copy(vals_hbm.at[pl.ds(col(j0), CH)], v0,
                                  sem_v0).wait()
            scatter_chunk(e0, v0)

            @pl.when(j0 + 2 < cnt)
            def _():
                pltpu.async_copy(edge_hbm.at[:, pl.ds(col(j0 + 2), CH)], e0,
                                 sem_e0)
                pltpu.async_copy(vals_hbm.at[pl.ds(col(j0 + 2), CH)], v0,
                                 sem_v0)

            @pl.when(j1 < cnt)
            def _():
                pltpu.make_async_copy(edge_hbm.at[:, pl.ds(col(j1), CH)], e1,
                                      sem_e1).wait()
                pltpu.make_async_copy(vals_hbm.at[pl.ds(col(j1), CH)], v1,
                                      sem_v1).wait()
                scatter_chunk(e1, v1)
            return carry

        lax.fori_loop(0, (cnt + 1) // 2, s_pair, 0)
        pltpu.sync_copy(big, part_hbm.at[wid])

    return edge_k


def _tc_body(nblk, part_ref, x_ref, batch_ref, wl_ref, bl_ref, wr_ref,
             wlin_ref, blin_ref, out_ref, sums, counts):
    i = pl.program_id(0)

    @pl.when(i == 0)
    def _init():
        sums[...] = jnp.zeros_like(sums)
        counts[...] = jnp.zeros_like(counts)

    agg = jnp.sum(part_ref[...], axis=0)
    xb = x_ref[0, :]
    h = jnp.maximum(
        agg[:, None] * wl_ref[0, :][None, :]
        + xb[:, None] * wr_ref[0, :][None, :]
        + bl_ref[0, :][None, :], 0.0)
    bb = batch_ref[0, :]
    gid = lax.broadcasted_iota(jnp.int32, (NUM_GRAPHS, 1), 0)
    onehot = (bb[None, :] == gid).astype(jnp.bfloat16)
    sums[...] += jnp.dot(onehot, h.astype(jnp.bfloat16),
                         preferred_element_type=jnp.float32)
    cnt = jnp.sum(onehot.astype(jnp.float32), axis=1, keepdims=True)
    counts[...] += jnp.broadcast_to(cnt, counts.shape)

    @pl.when(i == nblk - 1)
    def _fin():
        pooled = sums[...] / jnp.maximum(counts[...], 1.0)
        out_ref[...] = (jnp.dot(pooled, wlin_ref[...],
                                preferred_element_type=jnp.float32)
                        + blin_ref[0, :][None, :])


def _make_tc(N_pad, H, OUT, B):
    nblk = N_pad // B
    return pl.pallas_call(
        functools.partial(_tc_body, nblk),
        grid=(nblk,),
        in_specs=[
            pl.BlockSpec((_NW, B), lambda i: (0, i)),
            pl.BlockSpec((1, B), lambda i: (0, i)),
            pl.BlockSpec((1, B), lambda i: (0, i)),
            pl.BlockSpec((1, H), lambda i: (0, 0)),
            pl.BlockSpec((1, H), lambda i: (0, 0)),
            pl.BlockSpec((1, H), lambda i: (0, 0)),
            pl.BlockSpec((H, OUT), lambda i: (0, 0)),
            pl.BlockSpec((1, OUT), lambda i: (0, 0)),
        ],
        out_specs=pl.BlockSpec((NUM_GRAPHS, OUT), lambda i: (0, 0)),
        out_shape=jax.ShapeDtypeStruct((NUM_GRAPHS, OUT), jnp.float32),
        scratch_shapes=[
            pltpu.VMEM((NUM_GRAPHS, H), jnp.float32),
            pltpu.VMEM((NUM_GRAPHS, H), jnp.float32),
        ],
    )


def kernel(x, edge_index, batch, W_l, b_l, W_r, W_lin, b_lin):
    N = x.shape[0]
    E = edge_index.shape[1]
    H = W_l.shape[0]
    OUT = W_lin.shape[0]
    CH = 3200
    B = 4096
    N_pad = -(-N // B) * B

    xf = x.reshape(N)
    partials, _vals = _make_edge_kernel(N, N_pad, E, CH)(xf, edge_index)
    x_pad = jnp.pad(x.reshape(1, N), ((0, 0), (0, N_pad - N)))
    batch_pad = jnp.pad(batch.reshape(1, N), ((0, 0), (0, N_pad - N)),
                        constant_values=NUM_GRAPHS)
    out = _make_tc(N_pad, H, OUT, B)(
        partials,
        x_pad,
        batch_pad,
        W_l.reshape(1, H),
        b_l.reshape(1, H),
        W_r.reshape(1, H),
        W_lin.T,
        b_lin.reshape(1, OUT),
    )
    return out


# speedup vs baseline: 2.3515x; 1.1614x over previous
import functools

import jax
import jax.numpy as jnp
from jax import lax
from jax.experimental import pallas as pl
from jax.experimental.pallas import tpu as pltpu
from jax.experimental.pallas import tpu_sc as plsc

NUM_GRAPHS = 256

try:
    _info = plsc.get_sparse_core_info()
    _NC, _NS, _L = _info.num_cores, _info.num_subcores, _info.num_lanes
except Exception:
    _NC, _NS, _L = 2, 16, 16
_NW = _NC * _NS

_SC_PARAMS = pltpu.CompilerParams(needs_layout_passes=False)


def _worker_id():
    return lax.axis_index("s") * _NC + lax.axis_index("c")


def _make_edge_kernel(N, N_pad, E, CH):
    NCHT = E // CH
    NV = CH // _L
    mesh = plsc.VectorSubcoreMesh(core_axis_name="c", subcore_axis_name="s")

    @functools.partial(
        pl.kernel, mesh=mesh,
        out_type=(
            jax.ShapeDtypeStruct((_NW, N_pad), jnp.float32),
            jax.ShapeDtypeStruct((E,), jnp.float32),
        ),
        scratch_types=[
            pltpu.VMEM((N_pad,), jnp.float32),
            pltpu.VMEM((2, CH), jnp.int32),
            pltpu.VMEM((2, CH), jnp.int32),
            pltpu.VMEM((CH,), jnp.float32),
            pltpu.VMEM((CH,), jnp.float32),
            pltpu.SemaphoreType.DMA,
            pltpu.SemaphoreType.DMA,
            pltpu.SemaphoreType.DMA,
            pltpu.SemaphoreType.DMA,
            pltpu.SemaphoreType.DMA,
        ],
        compiler_params=_SC_PARAMS,
    )
    def edge_k(x_hbm, edge_hbm, part_hbm, vals_hbm, big, e0, e1, v0, v1,
               sem_x, sem_e0, sem_e1, sem_v0, sem_v1):
        wid = _worker_id()
        cnt = (NCHT - wid + _NW - 1) // _NW

        def col(j):
            return (wid + j * _NW) * CH

        pltpu.async_copy(x_hbm, big.at[pl.ds(0, N)], sem_x)
        pltpu.async_copy(edge_hbm.at[:, pl.ds(col(0), CH)], e0, sem_e0)
        pltpu.make_async_copy(x_hbm, big.at[pl.ds(0, N)], sem_x).wait()

        def gather_chunk(ebuf, vbuf):
            @plsc.parallel_loop(0, NV, unroll=8)
            def _(i):
                idx = ebuf[0, pl.ds(i * _L, _L)]
                vbuf[pl.ds(i * _L, _L)] = plsc.load_gather(big, [idx])

        def g_pair(p, carry):
            j0 = 2 * p
            j1 = j0 + 1

            @pl.when(j1 < cnt)
            def _():
                pltpu.async_copy(edge_hbm.at[:, pl.ds(col(j1), CH)], e1,
                                 sem_e1)
            pltpu.make_async_copy(edge_hbm.at[:, pl.ds(col(j0), CH)], e0,
                                  sem_e0).wait()
            @pl.when(j0 >= 2)
            def _():
                pltpu.make_async_copy(v0, vals_hbm.at[pl.ds(col(j0), CH)],
                                      sem_v0).wait()
            gather_chunk(e0, v0)
            pltpu.async_copy(v0, vals_hbm.at[pl.ds(col(j0), CH)], sem_v0)

            @pl.when(j0 + 2 < cnt)
            def _():
                pltpu.async_copy(edge_hbm.at[:, pl.ds(col(j0 + 2), CH)], e0,
                                 sem_e0)

            @pl.when(j1 < cnt)
            def _():
                pltpu.make_async_copy(edge_hbm.at[:, pl.ds(col(j1), CH)], e1,
                                      sem_e1).wait()
                @pl.when(j1 >= 2)
                def _():
                    pltpu.make_async_copy(
                        v1, vals_hbm.at[pl.ds(col(j1), CH)], sem_v1).wait()
                gather_chunk(e1, v1)
                pltpu.async_copy(v1, vals_hbm.at[pl.ds(col(j1), CH)], sem_v1)
            return carry

        lax.fori_loop(0, (cnt + 1) // 2, g_pair, 0)
        pltpu.make_async_copy(v0, vals_hbm.at[pl.ds(0, CH)], sem_v0).wait()

        @pl.when(cnt >= 2)
        def _():
            pltpu.make_async_copy(v1, vals_hbm.at[pl.ds(0, CH)], sem_v1).wait()

        pltpu.async_copy(edge_hbm.at[:, pl.ds(col(0), CH)], e0, sem_e0)
        pltpu.async_copy(vals_hbm.at[pl.ds(col(0), CH)], v0, sem_v0)

        def zero(i, c):
            for k in range(8):
                big[pl.ds(i * 8 * _L + k * _L, _L)] = jnp.zeros(
                    (_L,), jnp.float32)
            return c

        lax.fori_loop(0, N_pad // (8 * _L), zero, 0)

        def scatter_chunk(ebuf, vbuf):
            @plsc.parallel_loop(0, NV, unroll=8)
            def _(i):
                idx = ebuf[1, pl.ds(i * _L, _L)]
                v = vbuf[pl.ds(i * _L, _L)]
                plsc.addupdate_scatter(big, [idx], v)

        def s_pair(p, carry):
            j0 = 2 * p
            j1 = j0 + 1

            @pl.when(j1 < cnt)
            def _():
                pltpu.async_copy(edge_hbm.at[:, pl.ds(col(j1), CH)], e1,
                                 sem_e1)
                pltpu.async_copy(vals_hbm.at[pl.ds(col(j1), CH)], v1, sem_v1)
            pltpu.make_async_copy(edge_hbm.at[:, pl.ds(col(j0), CH)], e0,
                                  sem_e0).wait()
            pltpu.make_async_copy(vals_hbm.at[pl.ds(col(j0), CH)], v0,
                                  sem_v0).wait()
            scatter_chunk(e0, v0)

            @pl.when(j0 + 2 < cnt)
            def _():
                pltpu.async_copy(edge_hbm.at[:, pl.ds(col(j0 + 2), CH)], e0,
                                 sem_e0)
                pltpu.async_copy(vals_hbm.at[pl.ds(col(j0 + 2), CH)], v0,
                                 sem_v0)

            @pl.when(j1 < cnt)
            def _():
                pltpu.make_async_copy(edge_hbm.at[:, pl.ds(col(j1), CH)], e1,
                                      sem_e1).wait()
                pltpu.make_async_copy(vals_hbm.at[pl.ds(col(j1), CH)], v1,
                                      sem_v1).wait()
                scatter_chunk(e1, v1)
            return carry

        lax.fori_loop(0, (cnt + 1) // 2, s_pair, 0)
        pltpu.sync_copy(big, part_hbm.at[wid])

    return edge_k


def _tc_body(nblk, part_ref, x_ref, batch_ref, wl_ref, bl_ref, wr_ref,
             wlin_ref, blin_ref, out_ref, sums, counts):
    i = pl.program_id(0)

    @pl.when(i == 0)
    def _init():
        sums[...] = jnp.zeros_like(sums)
        counts[...] = jnp.zeros_like(counts)

    agg = jnp.sum(part_ref[...], axis=0)
    xb = x_ref[0, :]
    h = jnp.maximum(
        agg[:, None] * wl_ref[0, :][None, :]
        + xb[:, None] * wr_ref[0, :][None, :]
        + bl_ref[0, :][None, :], 0.0)
    bb = batch_ref[0, :]
    gid = lax.broadcasted_iota(jnp.int32, (NUM_GRAPHS, 1), 0)
    onehot = (bb[None, :] == gid).astype(jnp.bfloat16)
    sums[...] += jnp.dot(onehot, h.astype(jnp.bfloat16),
                         preferred_element_type=jnp.float32)
    cnt = jnp.sum(onehot.astype(jnp.float32), axis=1, keepdims=True)
    counts[...] += jnp.broadcast_to(cnt, counts.shape)

    @pl.when(i == nblk - 1)
    def _fin():
        pooled = sums[...] / jnp.maximum(counts[...], 1.0)
        out_ref[...] = (jnp.dot(pooled, wlin_ref[...],
                                preferred_element_type=jnp.float32)
                        + blin_ref[0, :][None, :])


def _make_tc(N_pad, H, OUT, B):
    nblk = N_pad // B
    return pl.pallas_call(
        functools.partial(_tc_body, nblk),
        grid=(nblk,),
        in_specs=[
            pl.BlockSpec((_NW, B), lambda i: (0, i)),
            pl.BlockSpec((1, B), lambda i: (0, i)),
            pl.BlockSpec((1, B), lambda i: (0, i)),
            pl.BlockSpec((1, H), lambda i: (0, 0)),
            pl.BlockSpec((1, H), lambda i: (0, 0)),
            pl.BlockSpec((1, H), lambda i: (0, 0)),
            pl.BlockSpec((H, OUT), lambda i: (0, 0)),
            pl.BlockSpec((1, OUT), lambda i: (0, 0)),
        ],
        out_specs=pl.BlockSpec((NUM_GRAPHS, OUT), lambda i: (0, 0)),
        out_shape=jax.ShapeDtypeStruct((NUM_GRAPHS, OUT), jnp.float32),
        scratch_shapes=[
            pltpu.VMEM((NUM_GRAPHS, H), jnp.float32),
            pltpu.VMEM((NUM_GRAPHS, H), jnp.float32),
        ],
    )


def kernel(x, edge_index, batch, W_l, b_l, W_r, W_lin, b_lin):
    N = x.shape[0]
    E = edge_index.shape[1]
    H = W_l.shape[0]
    OUT = W_lin.shape[0]
    CH = 3200
    B = 4096
    N_pad = -(-N // B) * B

    xf = x.reshape(N)
    partials, _vals = _make_edge_kernel(N, N_pad, E, CH)(xf, edge_index)
    x_pad = jnp.pad(x.reshape(1, N), ((0, 0), (0, N_pad - N)))
    batch_pad = jnp.pad(batch.reshape(1, N), ((0, 0), (0, N_pad - N)),
                        constant_values=NUM_GRAPHS)
    out = _make_tc(N_pad, H, OUT, B)(
        partials,
        x_pad,
        batch_pad,
        W_l.reshape(1, H),
        b_l.reshape(1, H),
        W_r.reshape(1, H),
        W_lin.T,
        b_lin.reshape(1, OUT),
    )
    return out
